# baseline (speedup 1.0000x reference)
import functools

import jax
import jax.numpy as jnp
from jax import lax
from jax.experimental import pallas as pl
from jax.experimental.pallas import tpu as pltpu
from jax.experimental.pallas import tpu_sc as plsc

_RANK = 64
_NC = 2
_NS = 16
_NW = _NC * _NS


def _sc_gather(ent, rel, x0, x1, x2):
    B = x0.shape[0]
    b_per_w = B // _NW
    mesh = plsc.VectorSubcoreMesh(core_axis_name="c", subcore_axis_name="s",
                                  num_cores=_NC)
    row = jax.ShapeDtypeStruct((B, _RANK), jnp.float32)

    @functools.partial(
        pl.kernel,
        mesh=mesh,
        out_type=(row, row, row),
        compiler_params=pltpu.CompilerParams(needs_layout_passes=False),
        scratch_types=[
            pltpu.VMEM((b_per_w,), jnp.int32),
            pltpu.VMEM((b_per_w,), jnp.int32),
            pltpu.VMEM((b_per_w,), jnp.int32),
            pltpu.VMEM((b_per_w, _RANK), jnp.float32),
            pltpu.VMEM((b_per_w, _RANK), jnp.float32),
            pltpu.VMEM((b_per_w, _RANK), jnp.float32),
            pltpu.SemaphoreType.DMA,
        ],
    )
    def gather_kernel(ent_hbm, rel_hbm, x0_hbm, x1_hbm, x2_hbm,
                      lhs_hbm, r_hbm, rhs_hbm,
                      v0, v1, v2, lv, rv, hv, sem):
        wid = lax.axis_index("s") * _NC + lax.axis_index("c")
        base = wid * b_per_w
        pltpu.sync_copy(x0_hbm.at[pl.ds(base, b_per_w)], v0)
        pltpu.sync_copy(x1_hbm.at[pl.ds(base, b_per_w)], v1)
        pltpu.sync_copy(x2_hbm.at[pl.ds(base, b_per_w)], v2)
        lane = lax.iota(jnp.int32, 16)
        for g in range(b_per_w // 16):
            w0 = v0[pl.ds(g * 16, 16)]
            w1 = v1[pl.ds(g * 16, 16)]
            w2 = v2[pl.ds(g * 16, 16)]
            z = jnp.zeros((16,), jnp.int32)
            for k in range(16):
                m = lane == jnp.full((16,), k, jnp.int32)
                s0 = jnp.sum(jnp.where(m, w0, z))
                s1 = jnp.sum(jnp.where(m, w1, z))
                s2 = jnp.sum(jnp.where(m, w2, z))
                kk = g * 16 + k
                pltpu.async_copy(ent_hbm.at[pl.ds(s0, 1)],
                                 lv.at[pl.ds(kk, 1)], sem)
                pltpu.async_copy(rel_hbm.at[pl.ds(s1, 1)],
                                 rv.at[pl.ds(kk, 1)], sem)
                pltpu.async_copy(ent_hbm.at[pl.ds(s2, 1)],
                                 hv.at[pl.ds(kk, 1)], sem)
        for kk in range(b_per_w):
            pltpu.make_async_copy(ent_hbm.at[pl.ds(0, 1)],
                                  lv.at[pl.ds(kk, 1)], sem).wait()
            pltpu.make_async_copy(rel_hbm.at[pl.ds(0, 1)],
                                  rv.at[pl.ds(kk, 1)], sem).wait()
            pltpu.make_async_copy(ent_hbm.at[pl.ds(0, 1)],
                                  hv.at[pl.ds(kk, 1)], sem).wait()
        pltpu.sync_copy(lv, lhs_hbm.at[pl.ds(base, b_per_w)])
        pltpu.sync_copy(rv, r_hbm.at[pl.ds(base, b_per_w)])
        pltpu.sync_copy(hv, rhs_hbm.at[pl.ds(base, b_per_w)])

    return gather_kernel(ent, rel, x0, x1, x2)


def _tc_scores_t(lhs, r, ent, eblk=5120):
    B = lhs.shape[0]
    n_ent = ent.shape[0]

    def mm_kernel(lhs_ref, r_ref, ent_ref, out_ref):
        q = lhs_ref[...] * r_ref[...]
        out_ref[...] = lax.dot_general(
            ent_ref[...], q, (((1,), (1,)), ((), ())),
            preferred_element_type=jnp.float32)

    return pl.pallas_call(
        mm_kernel,
        grid=(pl.cdiv(n_ent, eblk),),
        in_specs=[
            pl.BlockSpec((B, _RANK), lambda j: (0, 0)),
            pl.BlockSpec((B, _RANK), lambda j: (0, 0)),
            pl.BlockSpec((eblk, _RANK), lambda j: (j, 0)),
        ],
        out_specs=pl.BlockSpec((eblk, B), lambda j: (j, 0)),
        out_shape=jax.ShapeDtypeStruct((n_ent, B), jnp.float32),
        compiler_params=pltpu.CompilerParams(
            dimension_semantics=("arbitrary",)),
    )(lhs, r, ent)


@jax.jit
def kernel(x, ent, rel):
    x0 = x[:, 0]
    x1 = x[:, 1]
    x2 = x[:, 2]
    lhs, r, rhs = _sc_gather(ent, rel, x0, x1, x2)
    scores = _tc_scores_t(lhs, r, ent).T
    return (scores, (lhs, r, rhs))


# speedup vs baseline: 1.0049x; 1.0049x over previous
import functools

import jax
import jax.numpy as jnp
from jax import lax
from jax.experimental import pallas as pl
from jax.experimental.pallas import tpu as pltpu
from jax.experimental.pallas import tpu_sc as plsc

_RANK = 64
_NC = 2
_NS = 16
_NW = _NC * _NS


def _sc_gather(ent, rel, x):
    B = x.shape[0]
    b_per_w = B // _NW
    mesh = plsc.VectorSubcoreMesh(core_axis_name="c", subcore_axis_name="s",
                                  num_cores=_NC)
    row = jax.ShapeDtypeStruct((B, _RANK), jnp.float32)

    @functools.partial(
        pl.kernel,
        mesh=mesh,
        out_type=(row, row, row),
        compiler_params=pltpu.CompilerParams(needs_layout_passes=False),
        scratch_types=[
            pltpu.VMEM((b_per_w, 3), jnp.int32),
            pltpu.VMEM((b_per_w, _RANK), jnp.float32),
            pltpu.VMEM((b_per_w, _RANK), jnp.float32),
            pltpu.VMEM((b_per_w, _RANK), jnp.float32),
            pltpu.SemaphoreType.DMA,
        ],
    )
    def gather_kernel(ent_hbm, rel_hbm, x_hbm,
                      lhs_hbm, r_hbm, rhs_hbm,
                      xs, lv, rv, hv, sem):
        wid = lax.axis_index("s") * _NC + lax.axis_index("c")
        base = wid * b_per_w
        pltpu.sync_copy(x_hbm.at[pl.ds(base, b_per_w)], xs)
        lane = lax.iota(jnp.int32, 16)
        for g in range(b_per_w // 16):
            rows = lane + jnp.full((16,), g * 16, jnp.int32)
            w0 = plsc.load_gather(xs, [rows, jnp.zeros((16,), jnp.int32)])
            w1 = plsc.load_gather(xs, [rows, jnp.full((16,), 1, jnp.int32)])
            w2 = plsc.load_gather(xs, [rows, jnp.full((16,), 2, jnp.int32)])
            z = jnp.zeros((16,), jnp.int32)
            for k in range(16):
                m = lane == jnp.full((16,), k, jnp.int32)
                s0 = jnp.sum(jnp.where(m, w0, z))
                s1 = jnp.sum(jnp.where(m, w1, z))
                s2 = jnp.sum(jnp.where(m, w2, z))
                kk = g * 16 + k
                pltpu.async_copy(ent_hbm.at[pl.ds(s0, 1)],
                                 lv.at[pl.ds(kk, 1)], sem)
                pltpu.async_copy(rel_hbm.at[pl.ds(s1, 1)],
                                 rv.at[pl.ds(kk, 1)], sem)
                pltpu.async_copy(ent_hbm.at[pl.ds(s2, 1)],
                                 hv.at[pl.ds(kk, 1)], sem)
        for kk in range(b_per_w):
            pltpu.make_async_copy(ent_hbm.at[pl.ds(0, 1)],
                                  lv.at[pl.ds(kk, 1)], sem).wait()
            pltpu.make_async_copy(rel_hbm.at[pl.ds(0, 1)],
                                  rv.at[pl.ds(kk, 1)], sem).wait()
            pltpu.make_async_copy(ent_hbm.at[pl.ds(0, 1)],
                                  hv.at[pl.ds(kk, 1)], sem).wait()
        pltpu.sync_copy(lv, lhs_hbm.at[pl.ds(base, b_per_w)])
        pltpu.sync_copy(rv, r_hbm.at[pl.ds(base, b_per_w)])
        pltpu.sync_copy(hv, rhs_hbm.at[pl.ds(base, b_per_w)])

    return gather_kernel(ent, rel, x)


def _tc_scores_t(lhs, r, ent, eblk=4096):
    B = lhs.shape[0]
    n_ent = ent.shape[0]

    def mm_kernel(lhs_ref, r_ref, ent_ref, out_ref):
        q = lhs_ref[...] * r_ref[...]
        out_ref[...] = lax.dot_general(
            ent_ref[...], q, (((1,), (1,)), ((), ())),
            preferred_element_type=jnp.float32)

    return pl.pallas_call(
        mm_kernel,
        grid=(pl.cdiv(n_ent, eblk),),
        in_specs=[
            pl.BlockSpec((B, _RANK), lambda j: (0, 0)),
            pl.BlockSpec((B, _RANK), lambda j: (0, 0)),
            pl.BlockSpec((eblk, _RANK), lambda j: (j, 0)),
        ],
        out_specs=pl.BlockSpec((eblk, B), lambda j: (j, 0)),
        out_shape=jax.ShapeDtypeStruct((n_ent, B), jnp.float32),
        compiler_params=pltpu.CompilerParams(
            dimension_semantics=("arbitrary",)),
    )(lhs, r, ent)


@jax.jit
def kernel(x, ent, rel):
    lhs, r, rhs = _sc_gather(ent, rel, x)
    scores = _tc_scores_t(lhs, r, ent).T
    return (scores, (lhs, r, rhs))


# speedup vs baseline: 1.0248x; 1.0197x over previous
import functools

import jax
import jax.numpy as jnp
from jax import lax
from jax.experimental import pallas as pl
from jax.experimental.pallas import tpu as pltpu
from jax.experimental.pallas import tpu_sc as plsc

_RANK = 64
_NC = 2
_NS = 16
_NW = _NC * _NS


def _sc_gather(ent, rel, x):
    B = x.shape[0]
    b_per_w = B // _NW
    mesh = plsc.VectorSubcoreMesh(core_axis_name="c", subcore_axis_name="s",
                                  num_cores=_NC)
    row = jax.ShapeDtypeStruct((B, _RANK), jnp.float32)

    @functools.partial(
        pl.kernel,
        mesh=mesh,
        out_type=(row, row, row),
        compiler_params=pltpu.CompilerParams(needs_layout_passes=False),
        scratch_types=[
            pltpu.VMEM((b_per_w, 3), jnp.int32),
            pltpu.VMEM((b_per_w, _RANK), jnp.float32),
            pltpu.VMEM((b_per_w, _RANK), jnp.float32),
            pltpu.VMEM((b_per_w, _RANK), jnp.float32),
            pltpu.SemaphoreType.DMA,
        ],
    )
    def gather_kernel(ent_hbm, rel_hbm, x_hbm,
                      lhs_hbm, r_hbm, rhs_hbm,
                      xs, lv, rv, hv, sem):
        wid = lax.axis_index("s") * _NC + lax.axis_index("c")
        base = wid * b_per_w
        pltpu.sync_copy(x_hbm.at[pl.ds(base, b_per_w)], xs)
        lane = lax.iota(jnp.int32, 16)
        for g in range(b_per_w // 16):
            rows = lane + jnp.full((16,), g * 16, jnp.int32)
            w0 = plsc.load_gather(xs, [rows, jnp.zeros((16,), jnp.int32)])
            w1 = plsc.load_gather(xs, [rows, jnp.full((16,), 1, jnp.int32)])
            w2 = plsc.load_gather(xs, [rows, jnp.full((16,), 2, jnp.int32)])
            z = jnp.zeros((16,), jnp.int32)
            for k in range(16):
                m = lane == jnp.full((16,), k, jnp.int32)
                s0 = jnp.sum(jnp.where(m, w0, z))
                s1 = jnp.sum(jnp.where(m, w1, z))
                s2 = jnp.sum(jnp.where(m, w2, z))
                kk = g * 16 + k
                pltpu.async_copy(ent_hbm.at[pl.ds(s0, 1)],
                                 lv.at[pl.ds(kk, 1)], sem)
                pltpu.async_copy(rel_hbm.at[pl.ds(s1, 1)],
                                 rv.at[pl.ds(kk, 1)], sem)
                pltpu.async_copy(ent_hbm.at[pl.ds(s2, 1)],
                                 hv.at[pl.ds(kk, 1)], sem)
        for kk in range(b_per_w):
            pltpu.make_async_copy(ent_hbm.at[pl.ds(0, 1)],
                                  lv.at[pl.ds(kk, 1)], sem).wait()
            pltpu.make_async_copy(rel_hbm.at[pl.ds(0, 1)],
                                  rv.at[pl.ds(kk, 1)], sem).wait()
            pltpu.make_async_copy(ent_hbm.at[pl.ds(0, 1)],
                                  hv.at[pl.ds(kk, 1)], sem).wait()
        pltpu.sync_copy(lv, lhs_hbm.at[pl.ds(base, b_per_w)])
        pltpu.sync_copy(rv, r_hbm.at[pl.ds(base, b_per_w)])
        pltpu.sync_copy(hv, rhs_hbm.at[pl.ds(base, b_per_w)])

    return gather_kernel(ent, rel, x)


def _tc_scores_t(lhs, r, ent, eblk=4096):
    B = lhs.shape[0]
    ent_t = ent.T
    n_ent = ent_t.shape[1]

    def mm_kernel(lhs_ref, r_ref, ent_ref, out_ref):
        q = lhs_ref[...] * r_ref[...]
        out_ref[...] = lax.dot_general(
            ent_ref[...], q, (((0,), (1,)), ((), ())),
            preferred_element_type=jnp.float32)

    return pl.pallas_call(
        mm_kernel,
        grid=(pl.cdiv(n_ent, eblk),),
        in_specs=[
            pl.BlockSpec((B, _RANK), lambda j: (0, 0)),
            pl.BlockSpec((B, _RANK), lambda j: (0, 0)),
            pl.BlockSpec((_RANK, eblk), lambda j: (0, j)),
        ],
        out_specs=pl.BlockSpec((eblk, B), lambda j: (j, 0)),
        out_shape=jax.ShapeDtypeStruct((n_ent, B), jnp.float32),
        compiler_params=pltpu.CompilerParams(
            dimension_semantics=("arbitrary",)),
    )(lhs, r, ent_t)


@jax.jit
def kernel(x, ent, rel):
    lhs, r, rhs = _sc_gather(ent, rel, x)
    scores = _tc_scores_t(lhs, r, ent).T
    return (scores, (lhs, r, rhs))
